# Initial kernel scaffold; baseline (speedup 1.0000x reference)
import jax
import jax.numpy as jnp
from jax.experimental import pallas as pl


def kernel(inps, fw_adjs, bw_adjs, W_fw, b_fw, W_bw, b_bw, W1, b1):
    raise NotImplementedError("write your pallas kernel here")


# speedup vs baseline: 1677.9627x; 1677.9627x over previous
import functools

import jax
import jax.numpy as jnp
from jax.experimental import pallas as pl
from jax.experimental.pallas import tpu as pltpu

N = 1024
HID = 128
V = 2
F = HID // 2
BLOCK_D = 256

_T_DIMNUMS = (((0,), (0,)), ((), ()))


def _bignn_kernel(x_ref, afw_ref, abw_ref, wfw_ref, bfw_ref, wbw_ref,
                  bbw_ref, w1_ref, b1_ref, out_ref, hfw_ref, hbw_ref):
    j = pl.program_id(0)

    @pl.when(j == 0)
    def _compute_h():
        x = x_ref[...]
        for i in range(V):
            hfw_ref[pl.ds(i * N, N), :] = (
                jnp.dot(x, wfw_ref[i], preferred_element_type=jnp.float32)
                + bfw_ref[i:i + 1, :])
            hbw_ref[pl.ds(i * N, N), :] = (
                jnp.dot(x, wbw_ref[i], preferred_element_type=jnp.float32)
                + bbw_ref[i:i + 1, :])

    acc_parts = []
    for a_ref, h_ref in ((abw_ref, hbw_ref), (afw_ref, hfw_ref)):
        acc = None
        for i in range(V):
            a = a_ref[i].astype(jnp.float32)
            h = h_ref[pl.ds(i * N, N), :]
            agg = jax.lax.dot_general(a, h, _T_DIMNUMS,
                                      preferred_element_type=jnp.float32)
            r = jnp.maximum(agg, 0.0)
            acc = r if acc is None else acc + r
        acc_parts.append(acc)
    summed = jnp.concatenate(acc_parts, axis=-1)

    x_blk = x_ref[pl.ds(j * BLOCK_D, BLOCK_D), :]
    feats = (jnp.dot(jnp.maximum(summed, 0.0), w1_ref[...],
                     preferred_element_type=jnp.float32)
             + b1_ref[...] + x_blk)
    out_ref[...] = feats


@jax.jit
def kernel(inps, fw_adjs, bw_adjs, W_fw, b_fw, W_bw, b_bw, W1, b1):
    grid = N // BLOCK_D
    out = pl.pallas_call(
        _bignn_kernel,
        grid=(grid,),
        in_specs=[
            pl.BlockSpec((N, HID), lambda j: (0, 0)),
            pl.BlockSpec((V, N, BLOCK_D), lambda j: (0, 0, j)),
            pl.BlockSpec((V, N, BLOCK_D), lambda j: (0, 0, j)),
            pl.BlockSpec((V, HID, F), lambda j: (0, 0, 0)),
            pl.BlockSpec((V, F), lambda j: (0, 0)),
            pl.BlockSpec((V, HID, F), lambda j: (0, 0, 0)),
            pl.BlockSpec((V, F), lambda j: (0, 0)),
            pl.BlockSpec((HID, HID), lambda j: (0, 0)),
            pl.BlockSpec((1, HID), lambda j: (0, 0)),
        ],
        out_specs=pl.BlockSpec((BLOCK_D, HID), lambda j: (j, 0)),
        out_shape=jax.ShapeDtypeStruct((N, HID), jnp.float32),
        scratch_shapes=[
            pltpu.VMEM((V * N, F), jnp.float32),
            pltpu.VMEM((V * N, F), jnp.float32),
        ],
    )(inps, fw_adjs, bw_adjs, W_fw, b_fw, W_bw, b_bw, W1,
      b1.reshape(1, HID))
    return out
